# baseline (speedup 1.0000x reference)
import functools

import jax
import jax.numpy as jnp
from jax import lax
from jax.experimental import pallas as pl
from jax.experimental.pallas import tpu as pltpu
from jax.experimental.pallas import tpu_sc as plsc


def _largest_chunk(e, cap):
    for ch in range(cap - cap % 32, 31, -32):
        if e % ch == 0 and (e // ch) % 2 == 0:
            return ch
    return None


def _make_sc_kernel(n, d, e):
    info = plsc.get_sparse_core_info()
    num_tiles = info.num_cores * info.num_subcores
    assert d % num_tiles == 0
    cpt = d // num_tiles
    assert cpt % 2 == 0
    ppt = cpt // 2
    seg = cpt * n
    segp = ppt * n
    ch = _largest_chunk(e, 2048)
    assert ch is not None
    nch = e // ch
    chw = ch // 2

    mesh = plsc.VectorSubcoreMesh(core_axis_name="c", subcore_axis_name="s")

    @functools.partial(
        pl.kernel,
        out_type=[
            jax.ShapeDtypeStruct((d * n,), jnp.float32),
            jax.ShapeDtypeStruct((d * n,), jnp.float32),
        ],
        mesh=mesh,
        compiler_params=pltpu.CompilerParams(needs_layout_passes=False),
        scratch_types=[
            pltpu.VMEM((segp,), jnp.int32),
            pltpu.VMEM((seg,), jnp.float32),
            pltpu.VMEM((seg,), jnp.float32),
            pltpu.VMEM((chw,), jnp.int32),
            pltpu.VMEM((chw,), jnp.int32),
            pltpu.VMEM((chw,), jnp.int32),
            pltpu.VMEM((chw,), jnp.int32),
            pltpu.SemaphoreType.DMA,
            pltpu.SemaphoreType.DMA,
        ],
    )
    def k(a_hbm, b_hbm, ht_hbm, oin_hbm, oout_hbm, hloc, acc_in, acc_out,
          abuf0, bbuf0, abuf1, bbuf1, sem0, sem1):
        wid = lax.axis_index("s") * info.num_cores + lax.axis_index("c")
        base = wid * seg
        basep = wid * segp
        bufs = ((abuf0, bbuf0, sem0), (abuf1, bbuf1, sem1))

        def start(kk, par):
            ab, bb, sem = bufs[par]
            off = kk * chw
            pltpu.make_async_copy(a_hbm.at[pl.ds(off, chw)], ab, sem).start()
            pltpu.make_async_copy(b_hbm.at[pl.ds(off, chw)], bb, sem).start()

        def wait(par):
            ab, bb, sem = bufs[par]
            pltpu.make_async_copy(a_hbm.at[pl.ds(0, chw)], ab, sem).wait()
            pltpu.make_async_copy(b_hbm.at[pl.ds(0, chw)], bb, sem).wait()

        start(0, 0)
        start(1, 1)

        pltpu.sync_copy(ht_hbm.at[pl.ds(basep, segp)], hloc)

        zero = jnp.zeros((16,), jnp.float32)

        @plsc.parallel_loop(0, seg, 16, unroll=4)
        def zbody(i):
            acc_in[pl.ds(i, 16)] = zero
            acc_out[pl.ds(i, 16)] = zero

        def compute(par):
            ab, bb, _ = bufs[par]

            @plsc.parallel_loop(0, chw, 16, unroll=2)
            def group2(gw):
                aw = ab[pl.ds(gw, 16)]
                bw = bb[pl.ds(gw, 16)]
                a0, a1 = plsc.unpack(
                    plsc.bitcast(aw, jnp.int16),
                    format=plsc.PackFormat.INTERLEAVED)
                b0, b1 = plsc.unpack(
                    plsc.bitcast(bw, jnp.int16),
                    format=plsc.PackFormat.INTERLEAVED)

                def fetch(i16):
                    cols = []
                    for p in range(ppt):
                        w = plsc.load_gather(hloc, [i16 + p * n])
                        wb = plsc.bitcast(w, jnp.bfloat16)
                        lo, hi = plsc.unpack(
                            wb, format=plsc.PackFormat.INTERLEAVED)
                        cols += [lo, hi]
                    return cols

                for a16, b16 in ((a0, b0), (a1, b1)):
                    vas = fetch(a16)
                    vbs = fetch(b16)
                    ias = [a16 + c * n for c in range(cpt)]
                    ibs = [b16 + c * n for c in range(cpt)]
                    for c in range(cpt):
                        plsc.addupdate_scatter(acc_in, [ibs[c]], vas[c])
                        plsc.addupdate_scatter(acc_out, [ias[c]], vbs[c])

        def pair(p, _):
            for par in range(2):
                kk = 2 * p + par
                wait(par)
                compute(par)
                nxt = kk + 2

                @pl.when(nxt < nch)
                def _():
                    start(nxt, par)

            return 0

        lax.fori_loop(0, nch // 2, pair, 0)

        pltpu.sync_copy(acc_in, oin_hbm.at[pl.ds(base, seg)])
        pltpu.sync_copy(acc_out, oout_hbm.at[pl.ds(base, seg)])

    return k


@jax.jit
def kernel(edge_index, h):
    n, d = h.shape
    e = edge_index.shape[1]
    def pack_ids(row):
        r = row.astype(jnp.uint16).reshape(e // 32, 2, 16)
        r = jnp.swapaxes(r, 1, 2)
        return jax.lax.bitcast_convert_type(r, jnp.int32).reshape(-1)

    a = pack_ids(edge_index[0])
    b = pack_ids(edge_index[1])
    ht = jnp.swapaxes(h, 0, 1)
    hb = ht.astype(jnp.bfloat16).reshape(d // 2, 2, n)
    hb = jnp.swapaxes(hb, 1, 2)
    hp = jax.lax.bitcast_convert_type(hb, jnp.int32).reshape(-1)
    k = _make_sc_kernel(n, d, e)
    oin, oout = k(a, b, hp)
    h_in = jnp.swapaxes(oin.reshape(d, n), 0, 1)
    h_out = jnp.swapaxes(oout.reshape(d, n), 0, 1)
    return (h_in, h_out)


# speedup vs baseline: 1.0027x; 1.0027x over previous
import functools

import jax
import jax.numpy as jnp
from jax import lax
from jax.experimental import pallas as pl
from jax.experimental.pallas import tpu as pltpu
from jax.experimental.pallas import tpu_sc as plsc


def _largest_chunk(e, cap):
    for ch in range(cap - cap % 32, 31, -32):
        if e % ch == 0 and (e // ch) % 2 == 0:
            return ch
    return None


def _make_sc_kernel(n, d, e):
    info = plsc.get_sparse_core_info()
    num_tiles = info.num_cores * info.num_subcores
    assert d % num_tiles == 0
    cpt = d // num_tiles
    assert cpt % 2 == 0
    ppt = cpt // 2
    seg = cpt * n
    segp = ppt * n
    ch = _largest_chunk(e, 4096)
    assert ch is not None
    nch = e // ch
    chw = ch // 2

    mesh = plsc.VectorSubcoreMesh(core_axis_name="c", subcore_axis_name="s")

    @functools.partial(
        pl.kernel,
        out_type=[
            jax.ShapeDtypeStruct((d * n,), jnp.float32),
            jax.ShapeDtypeStruct((d * n,), jnp.float32),
        ],
        mesh=mesh,
        compiler_params=pltpu.CompilerParams(needs_layout_passes=False),
        scratch_types=[
            pltpu.VMEM((segp,), jnp.int32),
            pltpu.VMEM((seg,), jnp.float32),
            pltpu.VMEM((seg,), jnp.float32),
            pltpu.VMEM((chw,), jnp.int32),
            pltpu.VMEM((chw,), jnp.int32),
            pltpu.VMEM((chw,), jnp.int32),
            pltpu.VMEM((chw,), jnp.int32),
            pltpu.SemaphoreType.DMA,
            pltpu.SemaphoreType.DMA,
        ],
    )
    def k(a_hbm, b_hbm, ht_hbm, oin_hbm, oout_hbm, hloc, acc_in, acc_out,
          abuf0, bbuf0, abuf1, bbuf1, sem0, sem1):
        wid = lax.axis_index("s") * info.num_cores + lax.axis_index("c")
        base = wid * seg
        basep = wid * segp
        bufs = ((abuf0, bbuf0, sem0), (abuf1, bbuf1, sem1))

        def start(kk, par):
            ab, bb, sem = bufs[par]
            off = kk * chw
            pltpu.make_async_copy(a_hbm.at[pl.ds(off, chw)], ab, sem).start()
            pltpu.make_async_copy(b_hbm.at[pl.ds(off, chw)], bb, sem).start()

        def wait(par):
            ab, bb, sem = bufs[par]
            pltpu.make_async_copy(a_hbm.at[pl.ds(0, chw)], ab, sem).wait()
            pltpu.make_async_copy(b_hbm.at[pl.ds(0, chw)], bb, sem).wait()

        start(0, 0)
        start(1, 1)

        pltpu.sync_copy(ht_hbm.at[pl.ds(basep, segp)], hloc)

        zero = jnp.zeros((16,), jnp.float32)

        @plsc.parallel_loop(0, seg, 16, unroll=4)
        def zbody(i):
            acc_in[pl.ds(i, 16)] = zero
            acc_out[pl.ds(i, 16)] = zero

        def compute(par):
            ab, bb, _ = bufs[par]

            @plsc.parallel_loop(0, chw, 16, unroll=2)
            def group2(gw):
                aw = ab[pl.ds(gw, 16)]
                bw = bb[pl.ds(gw, 16)]
                a0, a1 = plsc.unpack(
                    plsc.bitcast(aw, jnp.int16),
                    format=plsc.PackFormat.INTERLEAVED)
                b0, b1 = plsc.unpack(
                    plsc.bitcast(bw, jnp.int16),
                    format=plsc.PackFormat.INTERLEAVED)

                def fetch(i16):
                    cols = []
                    for p in range(ppt):
                        w = plsc.load_gather(hloc, [i16 + p * n])
                        wb = plsc.bitcast(w, jnp.bfloat16)
                        lo, hi = plsc.unpack(
                            wb, format=plsc.PackFormat.INTERLEAVED)
                        cols += [lo, hi]
                    return cols

                for a16, b16 in ((a0, b0), (a1, b1)):
                    vas = fetch(a16)
                    vbs = fetch(b16)
                    ias = [a16 + c * n for c in range(cpt)]
                    ibs = [b16 + c * n for c in range(cpt)]
                    for c in range(cpt):
                        plsc.addupdate_scatter(acc_in, [ibs[c]], vas[c])
                        plsc.addupdate_scatter(acc_out, [ias[c]], vbs[c])

        def pair(p, _):
            for par in range(2):
                kk = 2 * p + par
                wait(par)
                compute(par)
                nxt = kk + 2

                @pl.when(nxt < nch)
                def _():
                    start(nxt, par)

            return 0

        lax.fori_loop(0, nch // 2, pair, 0)

        pltpu.sync_copy(acc_in, oin_hbm.at[pl.ds(base, seg)])
        pltpu.sync_copy(acc_out, oout_hbm.at[pl.ds(base, seg)])

    return k


@jax.jit
def kernel(edge_index, h):
    n, d = h.shape
    e = edge_index.shape[1]
    def pack_ids(row):
        r = row.astype(jnp.uint16).reshape(e // 32, 2, 16)
        r = jnp.swapaxes(r, 1, 2)
        return jax.lax.bitcast_convert_type(r, jnp.int32).reshape(-1)

    a = pack_ids(edge_index[0])
    b = pack_ids(edge_index[1])
    ht = jnp.swapaxes(h, 0, 1)
    hb = ht.astype(jnp.bfloat16).reshape(d // 2, 2, n)
    hb = jnp.swapaxes(hb, 1, 2)
    hp = jax.lax.bitcast_convert_type(hb, jnp.int32).reshape(-1)
    k = _make_sc_kernel(n, d, e)
    oin, oout = k(a, b, hp)
    h_in = jnp.swapaxes(oin.reshape(d, n), 0, 1)
    h_out = jnp.swapaxes(oout.reshape(d, n), 0, 1)
    return (h_in, h_out)


# speedup vs baseline: 1.3892x; 1.3854x over previous
import functools

import jax
import jax.numpy as jnp
from jax import lax
from jax.experimental import pallas as pl
from jax.experimental.pallas import tpu as pltpu
from jax.experimental.pallas import tpu_sc as plsc


def _largest_chunk(e, cap):
    for ch in range(cap - cap % 16, 15, -16):
        if e % ch == 0 and (e // ch) % 2 == 0:
            return ch
    return None


def _make_sc_kernel(n, d, e):
    info = plsc.get_sparse_core_info()
    num_tiles = info.num_cores * info.num_subcores
    assert d % num_tiles == 0
    cpt = d // num_tiles
    assert cpt % 2 == 0
    ppt = cpt // 2
    seg = cpt * n
    segp = ppt * n
    ch = _largest_chunk(e, 4096)
    assert ch is not None
    nch = e // ch

    mesh = plsc.VectorSubcoreMesh(core_axis_name="c", subcore_axis_name="s")

    @functools.partial(
        pl.kernel,
        out_type=[
            jax.ShapeDtypeStruct((d * n,), jnp.float32),
            jax.ShapeDtypeStruct((d * n,), jnp.float32),
        ],
        mesh=mesh,
        compiler_params=pltpu.CompilerParams(needs_layout_passes=False),
        scratch_types=[
            pltpu.VMEM((segp,), jnp.int32),
            pltpu.VMEM((seg,), jnp.float32),
            pltpu.VMEM((seg,), jnp.float32),
            pltpu.VMEM((ch,), jnp.int32),
            pltpu.VMEM((ch,), jnp.int32),
            pltpu.VMEM((ch,), jnp.int32),
            pltpu.VMEM((ch,), jnp.int32),
            pltpu.SemaphoreType.DMA,
            pltpu.SemaphoreType.DMA,
        ],
    )
    def k(a_hbm, b_hbm, ht_hbm, oin_hbm, oout_hbm, hloc, acc_in, acc_out,
          abuf0, bbuf0, abuf1, bbuf1, sem0, sem1):
        wid = lax.axis_index("s") * info.num_cores + lax.axis_index("c")
        base = wid * seg
        basep = wid * segp
        bufs = ((abuf0, bbuf0, sem0), (abuf1, bbuf1, sem1))

        def start(kk, par):
            ab, bb, sem = bufs[par]
            off = kk * ch
            pltpu.make_async_copy(a_hbm.at[pl.ds(off, ch)], ab, sem).start()
            pltpu.make_async_copy(b_hbm.at[pl.ds(off, ch)], bb, sem).start()

        def wait(par):
            ab, bb, sem = bufs[par]
            pltpu.make_async_copy(a_hbm.at[pl.ds(0, ch)], ab, sem).wait()
            pltpu.make_async_copy(b_hbm.at[pl.ds(0, ch)], bb, sem).wait()

        start(0, 0)
        start(1, 1)

        pltpu.sync_copy(ht_hbm.at[pl.ds(basep, segp)], hloc)

        zero = jnp.zeros((16,), jnp.float32)

        @plsc.parallel_loop(0, seg, 16, unroll=4)
        def zbody(i):
            acc_in[pl.ds(i, 16)] = zero
            acc_out[pl.ds(i, 16)] = zero

        def compute(par):
            ab, bb, _ = bufs[par]

            @plsc.parallel_loop(0, ch, 16, unroll=4)
            def group(g):
                a16 = ab[pl.ds(g, 16)]
                b16 = bb[pl.ds(g, 16)]

                def fetch(i16):
                    cols = []
                    for p in range(ppt):
                        w = plsc.load_gather(hloc, [i16 + p * n])
                        wb = plsc.bitcast(w, jnp.bfloat16)
                        lo, hi = plsc.unpack(
                            wb, format=plsc.PackFormat.INTERLEAVED)
                        cols += [lo, hi]
                    return cols

                vas = fetch(a16)
                vbs = fetch(b16)
                ias = [a16 + c * n for c in range(cpt)]
                ibs = [b16 + c * n for c in range(cpt)]
                for c in range(cpt):
                    plsc.addupdate_scatter(acc_in, [ibs[c]], vas[c])
                    plsc.addupdate_scatter(acc_out, [ias[c]], vbs[c])

        def pair(p, _):
            for par in range(2):
                kk = 2 * p + par
                wait(par)
                compute(par)
                nxt = kk + 2

                @pl.when(nxt < nch)
                def _():
                    start(nxt, par)

            return 0

        lax.fori_loop(0, nch // 2, pair, 0)

        pltpu.sync_copy(acc_in, oin_hbm.at[pl.ds(base, seg)])
        pltpu.sync_copy(acc_out, oout_hbm.at[pl.ds(base, seg)])

    return k


@jax.jit
def kernel(edge_index, h):
    n, d = h.shape
    e = edge_index.shape[1]
    a = edge_index[0]
    b = edge_index[1]
    ht = jnp.swapaxes(h, 0, 1)
    hb = ht.astype(jnp.bfloat16).reshape(d // 2, 2, n)
    hb = jnp.swapaxes(hb, 1, 2)
    hp = jax.lax.bitcast_convert_type(hb, jnp.int32).reshape(-1)
    k = _make_sc_kernel(n, d, e)
    oin, oout = k(a, b, hp)
    h_in = jnp.swapaxes(oin.reshape(d, n), 0, 1)
    h_out = jnp.swapaxes(oout.reshape(d, n), 0, 1)
    return (h_in, h_out)


# speedup vs baseline: 1.4506x; 1.0442x over previous
import functools

import jax
import jax.numpy as jnp
from jax import lax
from jax.experimental import pallas as pl
from jax.experimental.pallas import tpu as pltpu
from jax.experimental.pallas import tpu_sc as plsc


def _largest_chunk(e, cap):
    for step in (128, 16):
        for ch in range(cap - cap % step, step - 1, -step):
            if e % ch == 0 and (e // ch) % 2 == 0:
                return ch
    return None


def _make_sc_kernel(n, d, e):
    info = plsc.get_sparse_core_info()
    num_tiles = info.num_cores * info.num_subcores
    assert d % num_tiles == 0
    cpt = d // num_tiles
    assert cpt % 2 == 0
    ppt = cpt // 2
    seg = cpt * n
    segp = ppt * n
    ch = _largest_chunk(e, 8192)
    assert ch is not None
    nch = e // ch

    mesh = plsc.VectorSubcoreMesh(core_axis_name="c", subcore_axis_name="s")

    @functools.partial(
        pl.kernel,
        out_type=[
            jax.ShapeDtypeStruct((d * n,), jnp.float32),
            jax.ShapeDtypeStruct((d * n,), jnp.float32),
        ],
        mesh=mesh,
        compiler_params=pltpu.CompilerParams(needs_layout_passes=False),
        scratch_types=[
            pltpu.VMEM((segp,), jnp.int32),
            pltpu.VMEM((seg,), jnp.float32),
            pltpu.VMEM((seg,), jnp.float32),
            pltpu.VMEM((ch,), jnp.int32),
            pltpu.VMEM((ch,), jnp.int32),
            pltpu.VMEM((ch,), jnp.int32),
            pltpu.VMEM((ch,), jnp.int32),
            pltpu.SemaphoreType.DMA,
            pltpu.SemaphoreType.DMA,
            pltpu.SemaphoreType.DMA,
        ],
    )
    def k(ei_hbm, ht_hbm, oin_hbm, oout_hbm, hloc, acc_in, acc_out,
          abuf0, bbuf0, abuf1, bbuf1, sem0, sem1, sem2):
        wid = lax.axis_index("s") * info.num_cores + lax.axis_index("c")
        base = wid * seg
        basep = wid * segp
        bufs = ((abuf0, bbuf0, sem0), (abuf1, bbuf1, sem1))

        def start(kk, par):
            ab, bb, sem = bufs[par]
            off = kk * ch
            pltpu.make_async_copy(ei_hbm.at[0, pl.ds(off, ch)], ab, sem).start()
            pltpu.make_async_copy(ei_hbm.at[1, pl.ds(off, ch)], bb, sem).start()

        def wait(par):
            ab, bb, sem = bufs[par]
            pltpu.make_async_copy(ei_hbm.at[0, pl.ds(0, ch)], ab, sem).wait()
            pltpu.make_async_copy(ei_hbm.at[1, pl.ds(0, ch)], bb, sem).wait()

        start(0, 0)
        start(1, 1)
        hcopy = pltpu.make_async_copy(
            ht_hbm.at[pl.ds(basep, segp)], hloc, sem2)
        hcopy.start()

        zero = jnp.zeros((16,), jnp.float32)

        @plsc.parallel_loop(0, seg, 16, unroll=4)
        def zbody(i):
            acc_in[pl.ds(i, 16)] = zero
            acc_out[pl.ds(i, 16)] = zero

        hcopy.wait()

        def compute(par):
            ab, bb, _ = bufs[par]

            @plsc.parallel_loop(0, ch, 16, unroll=8)
            def group(g):
                a16 = ab[pl.ds(g, 16)]
                b16 = bb[pl.ds(g, 16)]

                def fetch(i16):
                    cols = []
                    for p in range(ppt):
                        w = plsc.load_gather(hloc, [i16 + p * n])
                        wb = plsc.bitcast(w, jnp.bfloat16)
                        lo, hi = plsc.unpack(
                            wb, format=plsc.PackFormat.INTERLEAVED)
                        cols += [lo, hi]
                    return cols

                vas = fetch(a16)
                vbs = fetch(b16)
                ias = [a16 + c * n for c in range(cpt)]
                ibs = [b16 + c * n for c in range(cpt)]
                for c in range(cpt):
                    plsc.addupdate_scatter(acc_in, [ibs[c]], vas[c])
                    plsc.addupdate_scatter(acc_out, [ias[c]], vbs[c])

        def pair(p, _):
            for par in range(2):
                kk = 2 * p + par
                wait(par)
                compute(par)
                nxt = kk + 2

                @pl.when(nxt < nch)
                def _():
                    start(nxt, par)

            return 0

        lax.fori_loop(0, nch // 2, pair, 0)

        c1 = pltpu.make_async_copy(acc_in, oin_hbm.at[pl.ds(base, seg)], sem1)
        c2 = pltpu.make_async_copy(acc_out, oout_hbm.at[pl.ds(base, seg)], sem2)
        c1.start()
        c2.start()
        c1.wait()
        c2.wait()

    return k


@jax.jit
def kernel(edge_index, h):
    n, d = h.shape
    e = edge_index.shape[1]
    hb = h.astype(jnp.bfloat16).reshape(n, d // 2, 2)
    hw = jax.lax.bitcast_convert_type(hb, jnp.int32)
    hp = jnp.swapaxes(hw, 0, 1).reshape(-1)
    k = _make_sc_kernel(n, d, e)
    oin, oout = k(edge_index, hp)
    h_in = jnp.swapaxes(oin.reshape(d, n), 0, 1)
    h_out = jnp.swapaxes(oout.reshape(d, n), 0, 1)
    return (h_in, h_out)
